# baseline (speedup 1.0000x reference)
import functools

import jax
import jax.numpy as jnp
from jax import lax
from jax.experimental import pallas as pl
from jax.experimental.pallas import tpu as pltpu
from jax.experimental.pallas import tpu_sc as plsc


_NC, _NS = 2, 16
_NW = _NC * _NS
_LANE = 128
_TROW = 8


def _detile_body(x_ref, o_ref):
    m, n_pk, _ = o_ref.shape
    for j in range(m):
        x = x_ref[:, j * _LANE:(j + 1) * _LANE]
        for p in range(n_pk // 4):
            lo = lax.bitcast_convert_type(
                x[8 * p:8 * p + 4, :].astype(jnp.bfloat16),
                jnp.uint16).astype(jnp.uint32)
            hi = lax.bitcast_convert_type(
                x[8 * p + 4:8 * p + 8, :].astype(jnp.bfloat16),
                jnp.uint16).astype(jnp.uint32)
            o_ref[j, pl.ds(4 * p, 4), :] = lax.bitcast_convert_type(
                lo | (hi << 16), jnp.int32)


@functools.lru_cache(maxsize=None)
def _make_sc_detile(vocab, n_feat):
    n_tiles = pl.cdiv(vocab, _LANE)
    last_full = n_tiles - 1
    per_w = pl.cdiv(n_tiles, _NW)
    kk = 8
    mesh = plsc.VectorSubcoreMesh(core_axis_name="c", subcore_axis_name="s")

    @functools.partial(
        pl.kernel,
        mesh=mesh,
        out_type=jax.ShapeDtypeStruct((n_tiles, n_feat, _LANE), jnp.float32),
        scratch_types=[
            pltpu.VMEM((kk, n_feat, _LANE), jnp.float32),
            pltpu.SemaphoreType.DMA,
            pltpu.SemaphoreType.DMA,
        ],
        compiler_params=pltpu.CompilerParams(use_tc_tiling_on_sc=True),
    )
    def sc_detile(tbl_hbm, out_hbm, slab_v, sem_i, sem_o):
        wid = lax.axis_index("s") * _NC + lax.axis_index("c")
        j0 = wid * per_w
        f0 = 32 - n_feat
        for cs in range(0, per_w, kk):
            cps = []
            for k in range(kk):
                j = jnp.minimum(j0 + cs + k, last_full)
                cps.append(pltpu.async_copy(
                    tbl_hbm.at[pl.ds(f0, n_feat), pl.ds(j * _LANE, _LANE)],
                    slab_v.at[k], sem_i))
            for cp in cps:
                cp.wait()
            cps = []
            for k in range(kk):
                j = jnp.minimum(j0 + cs + k, last_full)
                cps.append(pltpu.async_copy(slab_v.at[k], out_hbm.at[j],
                                            sem_o))
            for cp in cps:
                cp.wait()

    return sc_detile


@functools.lru_cache(maxsize=None)
def _make_detile(n_feat, vocab):
    n_tiles = pl.cdiv(vocab, _LANE)
    m = 1024
    n_jb = pl.cdiv(n_tiles, m)
    return pl.pallas_call(
        _detile_body,
        grid=(n_jb,),
        in_specs=[pl.BlockSpec((n_feat, _LANE * m), lambda jb: (0, jb))],
        out_specs=pl.BlockSpec((m, n_feat // 2, _LANE),
                               lambda jb: (jb, 0, 0)),
        out_shape=jax.ShapeDtypeStruct((n_tiles, n_feat // 2, _LANE),
                                       jnp.int32),
    )


@functools.lru_cache(maxsize=None)
def _make_gather(batch, n_pk, flat_len):
    b_per_w = batch // _NW
    n_q = b_per_w // _LANE
    mesh = plsc.VectorSubcoreMesh(core_axis_name="c", subcore_axis_name="s")
    out_sh = jax.ShapeDtypeStruct((n_pk, batch), jnp.int32)

    def build_flat_idx(idx_v, fidx_v):
        for q in range(n_q):
            for s in range(_LANE // 16):
                r = idx_v[q, pl.ds(s * 16, 16)]
                base = (r >> 7) * (n_pk * _LANE) + (r & (_LANE - 1))
                for c in range(n_pk):
                    fidx_v[c, q, pl.ds(s * 16, 16)] = base + _LANE * c

    @functools.partial(
        pl.kernel,
        mesh=mesh,
        out_type=out_sh,
        scratch_types=[
            pltpu.VMEM((n_q, _LANE), jnp.int32),
            pltpu.VMEM((n_pk, n_q, _LANE), jnp.int32),
            pltpu.VMEM((n_pk, b_per_w), jnp.int32),
            pltpu.SemaphoreType.DMA,
        ],
        compiler_params=pltpu.CompilerParams(use_tc_tiling_on_sc=False),
    )
    def gather_kernel(idx_hbm, tab_hbm, out_hbm, idx_v, fidx_v, rows_v, sem):
        wid = lax.axis_index("s") * _NC + lax.axis_index("c")
        base = wid * b_per_w
        for q in range(n_q):
            pltpu.sync_copy(idx_hbm.at[pl.ds(base + q * _LANE, _LANE)],
                            idx_v.at[q])
        build_flat_idx(idx_v, fidx_v)
        copies = []
        for c in range(n_pk):
            for q in range(n_q):
                copies.append(pltpu.async_copy(
                    tab_hbm.at[fidx_v.at[c, q]],
                    rows_v.at[c, pl.ds(q * _LANE, _LANE)], sem))
        for cp in copies:
            cp.wait()
        pltpu.sync_copy(rows_v, out_hbm.at[:, pl.ds(base, b_per_w)])

    return gather_kernel


@functools.lru_cache(maxsize=None)
def _make_gather_mixed(batch, n_pk8, n_f):
    b_per_w = batch // _NW
    n_q = b_per_w // _LANE
    mesh = plsc.VectorSubcoreMesh(core_axis_name="c", subcore_axis_name="s")

    @functools.partial(
        pl.kernel,
        mesh=mesh,
        out_type=(jax.ShapeDtypeStruct((n_pk8, batch), jnp.int32),
                  jax.ShapeDtypeStruct((n_f, batch), jnp.float32)),
        scratch_types=[
            pltpu.VMEM((n_q, _LANE), jnp.int32),
            pltpu.VMEM((n_pk8, n_q, _LANE), jnp.int32),
            pltpu.VMEM((n_f, n_q, _LANE), jnp.int32),
            pltpu.VMEM((n_pk8, b_per_w), jnp.int32),
            pltpu.VMEM((n_f, b_per_w), jnp.float32),
            pltpu.SemaphoreType.DMA,
        ],
        compiler_params=pltpu.CompilerParams(use_tc_tiling_on_sc=False),
    )
    def gather_mixed(idx_hbm, tab8_hbm, tabf_hbm, out8_hbm, outf_hbm,
                     idx_v, f8_v, ff_v, r8_v, rf_v, sem):
        wid = lax.axis_index("s") * _NC + lax.axis_index("c")
        base = wid * b_per_w
        for q in range(n_q):
            pltpu.sync_copy(idx_hbm.at[pl.ds(base + q * _LANE, _LANE)],
                            idx_v.at[q])
        for q in range(n_q):
            for s in range(_LANE // 16):
                r = idx_v[q, pl.ds(s * 16, 16)]
                tile = (r >> 7)
                off = r & (_LANE - 1)
                b8 = tile * (n_pk8 * _LANE) + off
                bf = tile * (n_f * _LANE) + off
                for c in range(n_pk8):
                    f8_v[c, q, pl.ds(s * 16, 16)] = b8 + _LANE * c
                for c in range(n_f):
                    ff_v[c, q, pl.ds(s * 16, 16)] = bf + _LANE * c
        copies = []
        for c in range(n_pk8):
            for q in range(n_q):
                copies.append(pltpu.async_copy(
                    tab8_hbm.at[f8_v.at[c, q]],
                    r8_v.at[c, pl.ds(q * _LANE, _LANE)], sem))
        for c in range(n_f):
            for q in range(n_q):
                copies.append(pltpu.async_copy(
                    tabf_hbm.at[ff_v.at[c, q]],
                    rf_v.at[c, pl.ds(q * _LANE, _LANE)], sem))
        for cp in copies:
            cp.wait()
        pltpu.sync_copy(r8_v, out8_hbm.at[:, pl.ds(base, b_per_w)])
        pltpu.sync_copy(rf_v, outf_hbm.at[:, pl.ds(base, b_per_w)])

    return gather_mixed


def _unpack(x):
    xu = lax.bitcast_convert_type(x, jnp.uint32)
    lo = lax.bitcast_convert_type(xu << 16, jnp.float32)
    hi = lax.bitcast_convert_type(xu & jnp.uint32(0xFFFF0000), jnp.float32)
    return lo, hi


def _mlp_body(ut8_ref, utf_ref, vt_ref, w1_ref, b1_ref, w2_ref, b2_ref,
              w3_ref, b3_ref, o_ref):
    n8 = ut8_ref.shape[0]
    nf = utf_ref.shape[0]
    npk = vt_ref.shape[0]
    ulo, uhi = _unpack(ut8_ref[...])
    vlo, vhi = _unpack(vt_ref[...])
    dn = (((0,), (0,)), ((), ()))
    s0, s1, s2, s3 = n8, 2 * n8, 2 * n8 + nf, 2 * n8 + nf + npk
    h = lax.dot_general(w1_ref[0:s0, :], ulo, dn,
                        preferred_element_type=jnp.float32)
    h = h + lax.dot_general(w1_ref[s0:s1, :], uhi, dn,
                            preferred_element_type=jnp.float32)
    h = h + lax.dot_general(w1_ref[s1:s2, :], utf_ref[...], dn,
                            preferred_element_type=jnp.float32)
    h = h + lax.dot_general(w1_ref[s2:s3, :], vlo, dn,
                            preferred_element_type=jnp.float32)
    h = h + lax.dot_general(w1_ref[s3:, :], vhi, dn,
                            preferred_element_type=jnp.float32)
    h = jnp.maximum(h + b1_ref[...][:, None], 0.0)
    h = lax.dot_general(w2_ref[...], h, dn,
                        preferred_element_type=jnp.float32)
    h = jnp.maximum(h + b2_ref[...][:, None], 0.0)
    w3 = w3_ref[...]
    o_ref[...] = jnp.sum(h * w3, axis=0) + b3_ref[...]


def kernel(user, item, user_emb, item_emb, W1, b1, W2, b2, W3, b3):
    batch = user.shape[0]
    vocab, emb_dim = user_emb.shape
    n_pk = emb_dim // 2
    n_hf = emb_dim // 2
    n_tiles = pl.cdiv(vocab, _LANE)
    u8_tab = _make_detile(emb_dim // 2, vocab)(user_emb.T)
    uf_tab = _make_sc_detile(vocab, n_hf)(user_emb.T)
    v_tab = _make_detile(emb_dim, vocab)(item_emb.T)
    ut8_g, utf_g = _make_gather_mixed(batch, n_pk // 2, n_hf)(
        user,
        u8_tab.reshape(n_tiles * (n_pk // 2) * _LANE),
        uf_tab.reshape(n_tiles * n_hf * _LANE))
    vt_g = _make_gather(batch, n_pk, n_tiles * n_pk * _LANE)(
        item, v_tab.reshape(n_tiles * n_pk * _LANE))
    p8 = [8 * (c2 // 4) + c2 % 4 for c2 in range(n_pk // 2)]
    p16 = [8 * (c2 // 4) + c2 % 4 for c2 in range(n_pk)]
    perm = (p8 + [p + 4 for p in p8]
            + list(range(emb_dim // 2, emb_dim))
            + [p + emb_dim for p in p16]
            + [p + emb_dim + 4 for p in p16])
    w1p = W1[jnp.asarray(perm, dtype=jnp.int32), :]
    return pl.pallas_call(
        _mlp_body,
        out_shape=jax.ShapeDtypeStruct((batch,), jnp.float32),
    )(ut8_g, utf_g, vt_g, w1p, b1, W2, b2, W3, b3)


# speedup vs baseline: 1.0478x; 1.0478x over previous
import functools

import jax
import jax.numpy as jnp
from jax import lax
from jax.experimental import pallas as pl
from jax.experimental.pallas import tpu as pltpu
from jax.experimental.pallas import tpu_sc as plsc


_NC, _NS = 2, 16
_NW = _NC * _NS
_LANE = 128
_TROW = 8


def _detile_body(x_ref, o_ref):
    m, n_pk, _ = o_ref.shape
    for j in range(m):
        x = x_ref[:, j * _LANE:(j + 1) * _LANE]
        for p in range(n_pk // 4):
            lo = lax.bitcast_convert_type(
                x[8 * p:8 * p + 4, :].astype(jnp.bfloat16),
                jnp.uint16).astype(jnp.uint32)
            hi = lax.bitcast_convert_type(
                x[8 * p + 4:8 * p + 8, :].astype(jnp.bfloat16),
                jnp.uint16).astype(jnp.uint32)
            o_ref[j, pl.ds(4 * p, 4), :] = lax.bitcast_convert_type(
                lo | (hi << 16), jnp.int32)


@functools.lru_cache(maxsize=None)
def _make_sc_detile(vocab, n_feat):
    n_tiles = pl.cdiv(vocab, _LANE)
    last_full = n_tiles - 1
    per_w = pl.cdiv(n_tiles, _NW)
    kk = 8
    mesh = plsc.VectorSubcoreMesh(core_axis_name="c", subcore_axis_name="s")

    @functools.partial(
        pl.kernel,
        mesh=mesh,
        out_type=jax.ShapeDtypeStruct((n_tiles, n_feat, _LANE), jnp.float32),
        scratch_types=[
            pltpu.VMEM((2, kk, n_feat, _LANE), jnp.float32),
            pltpu.SemaphoreType.DMA,
            pltpu.SemaphoreType.DMA,
        ],
        compiler_params=pltpu.CompilerParams(use_tc_tiling_on_sc=True),
    )
    def sc_detile(tbl_hbm, out_hbm, slab_v, sem_i, sem_o):
        wid = lax.axis_index("s") * _NC + lax.axis_index("c")
        j0 = wid * per_w
        f0 = 32 - n_feat
        nc = pl.cdiv(per_w, kk)

        def fire_in(c):
            cps = []
            for k in range(kk):
                j = jnp.minimum(j0 + c * kk + k, last_full)
                cps.append(pltpu.async_copy(
                    tbl_hbm.at[pl.ds(f0, n_feat), pl.ds(j * _LANE, _LANE)],
                    slab_v.at[c % 2, k], sem_i))
            return cps

        def fire_out(c):
            cps = []
            for k in range(kk):
                j = jnp.minimum(j0 + c * kk + k, last_full)
                cps.append(pltpu.async_copy(slab_v.at[c % 2, k],
                                            out_hbm.at[j], sem_o))
            return cps

        in_cps = {0: fire_in(0)}
        out_cps = {}
        for c in range(nc):
            if c >= 1:
                for cp in out_cps[c - 1]:
                    cp.wait()
            if c + 1 < nc:
                in_cps[c + 1] = fire_in(c + 1)
            for cp in in_cps[c]:
                cp.wait()
            out_cps[c] = fire_out(c)
        for cp in out_cps[nc - 1]:
            cp.wait()

    return sc_detile


@functools.lru_cache(maxsize=None)
def _make_detile(n_feat, vocab):
    n_tiles = pl.cdiv(vocab, _LANE)
    m = 1024
    n_jb = pl.cdiv(n_tiles, m)
    return pl.pallas_call(
        _detile_body,
        grid=(n_jb,),
        in_specs=[pl.BlockSpec((n_feat, _LANE * m), lambda jb: (0, jb))],
        out_specs=pl.BlockSpec((m, n_feat // 2, _LANE),
                               lambda jb: (jb, 0, 0)),
        out_shape=jax.ShapeDtypeStruct((n_tiles, n_feat // 2, _LANE),
                                       jnp.int32),
    )


@functools.lru_cache(maxsize=None)
def _make_gather(batch, n_pk, flat_len):
    b_per_w = batch // _NW
    n_q = b_per_w // _LANE
    mesh = plsc.VectorSubcoreMesh(core_axis_name="c", subcore_axis_name="s")
    out_sh = jax.ShapeDtypeStruct((n_pk, batch), jnp.int32)

    def build_flat_idx(idx_v, fidx_v):
        for q in range(n_q):
            for s in range(_LANE // 16):
                r = idx_v[q, pl.ds(s * 16, 16)]
                base = (r >> 7) * (n_pk * _LANE) + (r & (_LANE - 1))
                for c in range(n_pk):
                    fidx_v[c, q, pl.ds(s * 16, 16)] = base + _LANE * c

    @functools.partial(
        pl.kernel,
        mesh=mesh,
        out_type=out_sh,
        scratch_types=[
            pltpu.VMEM((n_q, _LANE), jnp.int32),
            pltpu.VMEM((n_pk, n_q, _LANE), jnp.int32),
            pltpu.VMEM((n_pk, b_per_w), jnp.int32),
            pltpu.SemaphoreType.DMA,
        ],
        compiler_params=pltpu.CompilerParams(use_tc_tiling_on_sc=False),
    )
    def gather_kernel(idx_hbm, tab_hbm, out_hbm, idx_v, fidx_v, rows_v, sem):
        wid = lax.axis_index("s") * _NC + lax.axis_index("c")
        base = wid * b_per_w
        for q in range(n_q):
            pltpu.sync_copy(idx_hbm.at[pl.ds(base + q * _LANE, _LANE)],
                            idx_v.at[q])
        build_flat_idx(idx_v, fidx_v)
        copies = []
        for c in range(n_pk):
            for q in range(n_q):
                copies.append(pltpu.async_copy(
                    tab_hbm.at[fidx_v.at[c, q]],
                    rows_v.at[c, pl.ds(q * _LANE, _LANE)], sem))
        for cp in copies:
            cp.wait()
        pltpu.sync_copy(rows_v, out_hbm.at[:, pl.ds(base, b_per_w)])

    return gather_kernel


@functools.lru_cache(maxsize=None)
def _make_gather_mixed(batch, n_pk8, n_f):
    b_per_w = batch // _NW
    n_q = b_per_w // _LANE
    mesh = plsc.VectorSubcoreMesh(core_axis_name="c", subcore_axis_name="s")

    @functools.partial(
        pl.kernel,
        mesh=mesh,
        out_type=(jax.ShapeDtypeStruct((n_pk8, batch), jnp.int32),
                  jax.ShapeDtypeStruct((n_f, batch), jnp.float32)),
        scratch_types=[
            pltpu.VMEM((n_q, _LANE), jnp.int32),
            pltpu.VMEM((n_pk8, n_q, _LANE), jnp.int32),
            pltpu.VMEM((n_f, n_q, _LANE), jnp.int32),
            pltpu.VMEM((n_pk8, b_per_w), jnp.int32),
            pltpu.VMEM((n_f, b_per_w), jnp.float32),
            pltpu.SemaphoreType.DMA,
        ],
        compiler_params=pltpu.CompilerParams(use_tc_tiling_on_sc=False),
    )
    def gather_mixed(idx_hbm, tab8_hbm, tabf_hbm, out8_hbm, outf_hbm,
                     idx_v, f8_v, ff_v, r8_v, rf_v, sem):
        wid = lax.axis_index("s") * _NC + lax.axis_index("c")
        base = wid * b_per_w
        for q in range(n_q):
            pltpu.sync_copy(idx_hbm.at[pl.ds(base + q * _LANE, _LANE)],
                            idx_v.at[q])
        for q in range(n_q):
            for s in range(_LANE // 16):
                r = idx_v[q, pl.ds(s * 16, 16)]
                tile = (r >> 7)
                off = r & (_LANE - 1)
                b8 = tile * (n_pk8 * _LANE) + off
                bf = tile * (n_f * _LANE) + off
                for c in range(n_pk8):
                    f8_v[c, q, pl.ds(s * 16, 16)] = b8 + _LANE * c
                for c in range(n_f):
                    ff_v[c, q, pl.ds(s * 16, 16)] = bf + _LANE * c
        copies = []
        for c in range(n_pk8):
            for q in range(n_q):
                copies.append(pltpu.async_copy(
                    tab8_hbm.at[f8_v.at[c, q]],
                    r8_v.at[c, pl.ds(q * _LANE, _LANE)], sem))
        for c in range(n_f):
            for q in range(n_q):
                copies.append(pltpu.async_copy(
                    tabf_hbm.at[ff_v.at[c, q]],
                    rf_v.at[c, pl.ds(q * _LANE, _LANE)], sem))
        for cp in copies:
            cp.wait()
        pltpu.sync_copy(r8_v, out8_hbm.at[:, pl.ds(base, b_per_w)])
        pltpu.sync_copy(rf_v, outf_hbm.at[:, pl.ds(base, b_per_w)])

    return gather_mixed


def _unpack(x):
    xu = lax.bitcast_convert_type(x, jnp.uint32)
    lo = lax.bitcast_convert_type(xu << 16, jnp.float32)
    hi = lax.bitcast_convert_type(xu & jnp.uint32(0xFFFF0000), jnp.float32)
    return lo, hi


def _mlp_body(ut8_ref, utf_ref, vt_ref, w1_ref, b1_ref, w2_ref, b2_ref,
              w3_ref, b3_ref, o_ref):
    n8 = ut8_ref.shape[0]
    nf = utf_ref.shape[0]
    npk = vt_ref.shape[0]
    ulo, uhi = _unpack(ut8_ref[...])
    vlo, vhi = _unpack(vt_ref[...])
    dn = (((0,), (0,)), ((), ()))
    s0, s1, s2, s3 = n8, 2 * n8, 2 * n8 + nf, 2 * n8 + nf + npk
    h = lax.dot_general(w1_ref[0:s0, :], ulo, dn,
                        preferred_element_type=jnp.float32)
    h = h + lax.dot_general(w1_ref[s0:s1, :], uhi, dn,
                            preferred_element_type=jnp.float32)
    h = h + lax.dot_general(w1_ref[s1:s2, :], utf_ref[...], dn,
                            preferred_element_type=jnp.float32)
    h = h + lax.dot_general(w1_ref[s2:s3, :], vlo, dn,
                            preferred_element_type=jnp.float32)
    h = h + lax.dot_general(w1_ref[s3:, :], vhi, dn,
                            preferred_element_type=jnp.float32)
    h = jnp.maximum(h + b1_ref[...][:, None], 0.0)
    h = lax.dot_general(w2_ref[...], h, dn,
                        preferred_element_type=jnp.float32)
    h = jnp.maximum(h + b2_ref[...][:, None], 0.0)
    w3 = w3_ref[...]
    o_ref[...] = jnp.sum(h * w3, axis=0) + b3_ref[...]


def kernel(user, item, user_emb, item_emb, W1, b1, W2, b2, W3, b3):
    batch = user.shape[0]
    vocab, emb_dim = user_emb.shape
    n_pk = emb_dim // 2
    n_hf = emb_dim // 2
    n_tiles = pl.cdiv(vocab, _LANE)
    u8_tab = _make_detile(emb_dim // 2, vocab)(user_emb.T)
    uf_tab = _make_sc_detile(vocab, n_hf)(user_emb.T)
    v_tab = _make_detile(emb_dim, vocab)(item_emb.T)
    ut8_g, utf_g = _make_gather_mixed(batch, n_pk // 2, n_hf)(
        user,
        u8_tab.reshape(n_tiles * (n_pk // 2) * _LANE),
        uf_tab.reshape(n_tiles * n_hf * _LANE))
    vt_g = _make_gather(batch, n_pk, n_tiles * n_pk * _LANE)(
        item, v_tab.reshape(n_tiles * n_pk * _LANE))
    p8 = [8 * (c2 // 4) + c2 % 4 for c2 in range(n_pk // 2)]
    p16 = [8 * (c2 // 4) + c2 % 4 for c2 in range(n_pk)]
    perm = (p8 + [p + 4 for p in p8]
            + list(range(emb_dim // 2, emb_dim))
            + [p + emb_dim for p in p16]
            + [p + emb_dim + 4 for p in p16])
    w1p = W1[jnp.asarray(perm, dtype=jnp.int32), :]
    return pl.pallas_call(
        _mlp_body,
        out_shape=jax.ShapeDtypeStruct((batch,), jnp.float32),
    )(ut8_g, utf_g, vt_g, w1p, b1, W2, b2, W3, b3)


# speedup vs baseline: 1.1974x; 1.1427x over previous
import functools

import jax
import jax.numpy as jnp
from jax import lax
from jax.experimental import pallas as pl
from jax.experimental.pallas import tpu as pltpu
from jax.experimental.pallas import tpu_sc as plsc


_NC, _NS = 2, 16
_NW = _NC * _NS
_LANE = 128
_TROW = 8


def _detile_body(x_ref, o_ref):
    m, n_pk, _ = o_ref.shape
    for j in range(m):
        x = x_ref[:, j * _LANE:(j + 1) * _LANE]
        for p in range(n_pk // 4):
            lo = lax.bitcast_convert_type(
                x[8 * p:8 * p + 4, :].astype(jnp.bfloat16),
                jnp.uint16).astype(jnp.uint32)
            hi = lax.bitcast_convert_type(
                x[8 * p + 4:8 * p + 8, :].astype(jnp.bfloat16),
                jnp.uint16).astype(jnp.uint32)
            o_ref[j, pl.ds(4 * p, 4), :] = lax.bitcast_convert_type(
                lo | (hi << 16), jnp.int32)


@functools.lru_cache(maxsize=None)
def _make_detile(emb_dim, vocab):
    n_tiles = pl.cdiv(vocab, _LANE)
    m = 1024
    n_jb = pl.cdiv(n_tiles, m)
    return pl.pallas_call(
        _detile_body,
        grid=(n_jb,),
        in_specs=[pl.BlockSpec((emb_dim, _LANE * m), lambda jb: (0, jb))],
        out_specs=pl.BlockSpec((m, emb_dim // 2, _LANE),
                               lambda jb: (jb, 0, 0)),
        out_shape=jax.ShapeDtypeStruct((n_tiles, emb_dim // 2, _LANE),
                                       jnp.int32),
    )


@functools.lru_cache(maxsize=None)
def _make_gather(batch, n_pk, flat_len):
    b_per_w = batch // _NW
    n_q = b_per_w // _LANE
    mesh = plsc.VectorSubcoreMesh(core_axis_name="c", subcore_axis_name="s")
    out_sh = jax.ShapeDtypeStruct((n_pk, batch), jnp.int32)

    def build_flat_idx(idx_v, fidx_v):
        for q in range(n_q):
            for s in range(_LANE // 16):
                r = idx_v[q, pl.ds(s * 16, 16)]
                base = (r >> 7) * (n_pk * _LANE) + (r & (_LANE - 1))
                for c in range(n_pk):
                    fidx_v[c, q, pl.ds(s * 16, 16)] = base + _LANE * c

    @functools.partial(
        pl.kernel,
        mesh=mesh,
        out_type=out_sh,
        scratch_types=[
            pltpu.VMEM((n_q, _LANE), jnp.int32),
            pltpu.VMEM((n_pk, n_q, _LANE), jnp.int32),
            pltpu.VMEM((n_pk, b_per_w), jnp.int32),
            pltpu.SemaphoreType.DMA,
        ],
        compiler_params=pltpu.CompilerParams(use_tc_tiling_on_sc=False),
    )
    def gather_kernel(idx_hbm, tab_hbm, out_hbm, idx_v, fidx_v, rows_v, sem):
        wid = lax.axis_index("s") * _NC + lax.axis_index("c")
        base = wid * b_per_w
        for q in range(n_q):
            pltpu.sync_copy(idx_hbm.at[pl.ds(base + q * _LANE, _LANE)],
                            idx_v.at[q])
        build_flat_idx(idx_v, fidx_v)
        copies = []
        for c in range(n_pk):
            for q in range(n_q):
                copies.append(pltpu.async_copy(
                    tab_hbm.at[fidx_v.at[c, q]],
                    rows_v.at[c, pl.ds(q * _LANE, _LANE)], sem))
        for cp in copies:
            cp.wait()
        pltpu.sync_copy(rows_v, out_hbm.at[:, pl.ds(base, b_per_w)])

    return gather_kernel


def _unpack(x):
    xu = lax.bitcast_convert_type(x, jnp.uint32)
    lo = lax.bitcast_convert_type(xu << 16, jnp.float32)
    hi = lax.bitcast_convert_type(xu & jnp.uint32(0xFFFF0000), jnp.float32)
    return lo, hi


def _mlp_body(ut_ref, vt_ref, w1_ref, b1_ref, w2_ref, b2_ref, w3_ref, b3_ref,
              o_ref):
    n_pk = ut_ref.shape[0]
    ulo, uhi = _unpack(ut_ref[...])
    vlo, vhi = _unpack(vt_ref[...])
    dn = (((0,), (0,)), ((), ()))
    h = lax.dot_general(w1_ref[0:n_pk, :], ulo, dn,
                        preferred_element_type=jnp.float32)
    h = h + lax.dot_general(w1_ref[n_pk:2 * n_pk, :], uhi, dn,
                            preferred_element_type=jnp.float32)
    h = h + lax.dot_general(w1_ref[2 * n_pk:3 * n_pk, :], vlo, dn,
                            preferred_element_type=jnp.float32)
    h = h + lax.dot_general(w1_ref[3 * n_pk:, :], vhi, dn,
                            preferred_element_type=jnp.float32)
    h = jnp.maximum(h + b1_ref[...][:, None], 0.0)
    h = lax.dot_general(w2_ref[...], h, dn,
                        preferred_element_type=jnp.float32)
    h = jnp.maximum(h + b2_ref[...][:, None], 0.0)
    w3 = w3_ref[...]
    o_ref[...] = jnp.sum(h * w3, axis=0) + b3_ref[...]


def kernel(user, item, user_emb, item_emb, W1, b1, W2, b2, W3, b3):
    batch = user.shape[0]
    vocab, emb_dim = user_emb.shape
    n_pk = emb_dim // 2
    detile = _make_detile(emb_dim, vocab)
    u_tab = detile(user_emb.T)
    v_tab = detile(item_emb.T)
    flat_len = u_tab.shape[0] * n_pk * _LANE
    gather = _make_gather(batch, n_pk, flat_len)
    ut_g = gather(user, u_tab.reshape(flat_len))
    vt_g = gather(item, v_tab.reshape(flat_len))
    perm_lo = [8 * (c2 // 4) + c2 % 4 for c2 in range(n_pk)]
    perm = (perm_lo + [p + 4 for p in perm_lo]
            + [p + emb_dim for p in perm_lo]
            + [p + emb_dim + 4 for p in perm_lo])
    w1p = W1[jnp.asarray(perm, dtype=jnp.int32), :]
    return pl.pallas_call(
        _mlp_body,
        out_shape=jax.ShapeDtypeStruct((batch,), jnp.float32),
    )(ut_g, vt_g, w1p, b1, W2, b2, W3, b3)
